# Initial kernel scaffold; baseline (speedup 1.0000x reference)
import functools

import jax
import jax.numpy as jnp
from jax import lax
from jax.experimental import pallas as pl
from jax.experimental.pallas import tpu as pltpu
from jax.experimental.pallas import tpu_sc as plsc

EPS = 1e-5
F32 = jnp.float32


def _make_sc_gather(total_idx):
    info = plsc.get_sparse_core_info()
    nc, ns = info.num_cores, info.num_subcores
    nw = nc * ns
    assert total_idx % nw == 0
    per_w = total_idx // nw
    chunk = 2000
    assert per_w % chunk == 0 and chunk % 8 == 0
    nch = per_w // chunk
    mesh = plsc.VectorSubcoreMesh(core_axis_name="c", subcore_axis_name="s")

    @functools.partial(
        pl.kernel,
        out_type=jax.ShapeDtypeStruct((total_idx, 16), F32),
        mesh=mesh,
        scratch_types=[
            pltpu.VMEM((chunk,), jnp.int32),
            pltpu.VMEM((chunk,), jnp.int32),
            pltpu.VMEM((chunk, 16), F32),
            pltpu.VMEM((chunk, 16), F32),
            pltpu.SemaphoreType.DMA,
            pltpu.SemaphoreType.DMA,
        ],
    )
    def sc_gather(p_hbm, idx_hbm, g_hbm, idx0, idx1, buf0, buf1, sem0, sem1):
        wid = lax.axis_index("s") * nc + lax.axis_index("c")
        base = wid * per_w
        idxs, bufs, sems = [idx0, idx1], [buf0, buf1], [sem0, sem1]
        copies = [None, None]
        pltpu.sync_copy(idx_hbm.at[pl.ds(base, chunk)], idx0)
        copies[0] = pltpu.async_copy(p_hbm.at[idx0], buf0, sem0)
        for c in range(nch):
            cur = c & 1
            nxt = (c + 1) & 1
            if c + 1 < nch:
                pltpu.sync_copy(
                    idx_hbm.at[pl.ds(base + (c + 1) * chunk, chunk)], idxs[nxt]
                )
                copies[nxt] = pltpu.async_copy(
                    p_hbm.at[idxs[nxt]], bufs[nxt], sems[nxt]
                )
            copies[cur].wait()
            pltpu.sync_copy(bufs[cur], g_hbm.at[pl.ds(base + c * chunk, chunk)])

    return sc_gather


def _k1_body(atom_ref, ws_ref, wn_ref, bb_ref, s_ref, p_ref):
    a = atom_ref[...]
    s_ref[...] = (
        jnp.dot(a, ws_ref[...], preferred_element_type=F32) + bb_ref[...]
    )
    p_ref[...] = jnp.dot(a, wn_ref[...], preferred_element_type=F32)


def _k2_body(nb, g_ref, bond_ref, s_ref, w128_ref, e_ref, et_ref,
             bf_ref, pooled_ref, stats_ref):
    b = jnp.dot(bond_ref[...], w128_ref[...], preferred_element_type=F32)
    s128 = jnp.dot(s_ref[...], e_ref[...], preferred_element_type=F32)
    sfull = jnp.broadcast_to(s128[:, None, :], (nb, 4, 128)).reshape(nb * 4, 128)
    bf = jnp.tanh(g_ref[...] + b + sfull)
    bf_ref[...] = bf
    sum4 = jnp.sum(bf.reshape(nb, 4, 128), axis=1)
    pooled_ref[...] = jnp.dot(
        sum4, et_ref[...], preferred_element_type=F32
    ) * (1.0 / 32.0)
    part = jnp.concatenate(
        [jnp.sum(bf, axis=0, keepdims=True),
         jnp.sum(bf * bf, axis=0, keepdims=True)], axis=0)

    @pl.when(pl.program_id(0) == 0)
    def _():
        stats_ref[...] = part

    @pl.when(pl.program_id(0) != 0)
    def _():
        stats_ref[...] += part


def _k3_body(bf_ref, pooled_ref, atom_ref, wap_ref, waa_ref, ba_ref,
             aff16_ref, aff128_ref, bfout_ref, af_ref, stats_ref):
    a1_16 = aff16_ref[0:1, :]
    c1_16 = aff16_ref[1:2, :]
    a1_128 = aff128_ref[0:1, :]
    c1_128 = aff128_ref[1:2, :]
    bfout_ref[...] = bf_ref[...] * a1_128 + c1_128
    pooled_n = pooled_ref[...] * a1_16 + c1_16
    h = (
        jnp.dot(pooled_n, wap_ref[...], preferred_element_type=F32)
        + jnp.dot(atom_ref[...], waa_ref[...], preferred_element_type=F32)
        + ba_ref[...]
    )
    af = jnp.tanh(h)
    af_ref[...] = af
    part = jnp.concatenate(
        [jnp.sum(af, axis=0, keepdims=True),
         jnp.sum(af * af, axis=0, keepdims=True)], axis=0)

    @pl.when(pl.program_id(0) == 0)
    def _():
        stats_ref[...] = part

    @pl.when(pl.program_id(0) != 0)
    def _():
        stats_ref[...] += part


def _k4_body(af_ref, aff_ref, out_ref):
    out_ref[...] = af_ref[...] * aff_ref[0:1, :] + aff_ref[1:2, :]


def _affine_from_stats(stats16, cnt, gamma, beta):
    mu = stats16[0] / cnt
    var = stats16[1] / cnt - mu * mu
    a = gamma * lax.rsqrt(var + EPS)
    c = beta - mu * a
    return a, c


def kernel(nbr_idx, atom_fea, bond_fea, Wb, bb, Wa, ba, g1, be1, g2, be2):
    n, nnn = nbr_idx.shape
    a_fea = atom_fea.shape[1]
    b_fea = bond_fea.shape[2]
    assert (n, nnn, a_fea, b_fea) == (10000, 32, 128, 16)
    tot = n * nnn
    rows_p = tot * b_fea // 128
    nb = 1000
    grid = n // nb
    rbp = nb * nnn * b_fea // 128

    idx = nbr_idx.reshape(-1).astype(jnp.int32)
    Wb_self, Wb_nbr, Wb_bond = Wb[:a_fea], Wb[a_fea:2 * a_fea], Wb[2 * a_fea:]
    W128 = jnp.kron(jnp.eye(8, dtype=F32), Wb_bond)
    E = (jnp.arange(128)[None, :] % 16 == jnp.arange(16)[:, None]).astype(F32)
    bond_p = bond_fea.reshape(rows_p, 128)

    s_tab, p_tab = pl.pallas_call(
        _k1_body,
        grid=(grid,),
        in_specs=[
            pl.BlockSpec((nb, a_fea), lambda i: (i, 0)),
            pl.BlockSpec((a_fea, b_fea), lambda i: (0, 0)),
            pl.BlockSpec((a_fea, b_fea), lambda i: (0, 0)),
            pl.BlockSpec((1, b_fea), lambda i: (0, 0)),
        ],
        out_specs=[
            pl.BlockSpec((nb, b_fea), lambda i: (i, 0)),
            pl.BlockSpec((nb, b_fea), lambda i: (i, 0)),
        ],
        out_shape=[
            jax.ShapeDtypeStruct((n, b_fea), F32),
            jax.ShapeDtypeStruct((n, b_fea), F32),
        ],
    )(atom_fea, Wb_self, Wb_nbr, bb.reshape(1, b_fea))

    g_rows = _make_sc_gather(tot)(p_tab, idx)
    g_p = g_rows.reshape(rows_p, 128)

    bf_raw_p, pooled, stats1 = pl.pallas_call(
        functools.partial(_k2_body, nb),
        grid=(grid,),
        in_specs=[
            pl.BlockSpec((rbp, 128), lambda i: (i, 0)),
            pl.BlockSpec((rbp, 128), lambda i: (i, 0)),
            pl.BlockSpec((nb, b_fea), lambda i: (i, 0)),
            pl.BlockSpec((128, 128), lambda i: (0, 0)),
            pl.BlockSpec((b_fea, 128), lambda i: (0, 0)),
            pl.BlockSpec((128, b_fea), lambda i: (0, 0)),
        ],
        out_specs=[
            pl.BlockSpec((rbp, 128), lambda i: (i, 0)),
            pl.BlockSpec((nb, b_fea), lambda i: (i, 0)),
            pl.BlockSpec((2, 128), lambda i: (0, 0)),
        ],
        out_shape=[
            jax.ShapeDtypeStruct((rows_p, 128), F32),
            jax.ShapeDtypeStruct((n, b_fea), F32),
            jax.ShapeDtypeStruct((2, 128), F32),
        ],
    )(g_p, bond_p, s_tab, W128, E, E.T)

    st16 = stats1.reshape(2, 8, b_fea).sum(axis=1)
    a1, c1 = _affine_from_stats(st16, float(tot), g1, be1)
    aff16 = jnp.stack([a1, c1])
    aff128 = jnp.stack([jnp.tile(a1, 8), jnp.tile(c1, 8)])

    bf_out_p, af_raw, stats2 = pl.pallas_call(
        _k3_body,
        grid=(grid,),
        in_specs=[
            pl.BlockSpec((rbp, 128), lambda i: (i, 0)),
            pl.BlockSpec((nb, b_fea), lambda i: (i, 0)),
            pl.BlockSpec((nb, a_fea), lambda i: (i, 0)),
            pl.BlockSpec((b_fea, a_fea), lambda i: (0, 0)),
            pl.BlockSpec((a_fea, a_fea), lambda i: (0, 0)),
            pl.BlockSpec((1, a_fea), lambda i: (0, 0)),
            pl.BlockSpec((2, b_fea), lambda i: (0, 0)),
            pl.BlockSpec((2, 128), lambda i: (0, 0)),
        ],
        out_specs=[
            pl.BlockSpec((rbp, 128), lambda i: (i, 0)),
            pl.BlockSpec((nb, a_fea), lambda i: (i, 0)),
            pl.BlockSpec((2, a_fea), lambda i: (0, 0)),
        ],
        out_shape=[
            jax.ShapeDtypeStruct((rows_p, 128), F32),
            jax.ShapeDtypeStruct((n, a_fea), F32),
            jax.ShapeDtypeStruct((2, a_fea), F32),
        ],
    )(bf_raw_p, pooled, atom_fea, Wa[:b_fea], Wa[b_fea:],
      ba.reshape(1, a_fea), aff16, aff128)

    a2, c2 = _affine_from_stats(stats2, float(n), g2, be2)

    af_out = pl.pallas_call(
        _k4_body,
        grid=(grid,),
        in_specs=[
            pl.BlockSpec((nb, a_fea), lambda i: (i, 0)),
            pl.BlockSpec((2, a_fea), lambda i: (0, 0)),
        ],
        out_specs=pl.BlockSpec((nb, a_fea), lambda i: (i, 0)),
        out_shape=jax.ShapeDtypeStruct((n, a_fea), F32),
    )(af_raw, jnp.stack([a2, c2]))

    return af_out, bf_out_p.reshape(n, nnn, b_fea)


# speedup vs baseline: 3.1428x; 3.1428x over previous
import functools

import jax
import jax.numpy as jnp
from jax import lax
from jax.experimental import pallas as pl
from jax.experimental.pallas import tpu as pltpu
from jax.experimental.pallas import tpu_sc as plsc

EPS = 1e-5
F32 = jnp.float32


def _make_sc_gather(total_idx):
    info = plsc.get_sparse_core_info()
    nc, ns = info.num_cores, info.num_subcores
    nw = nc * ns
    assert total_idx % nw == 0
    per_w = total_idx // nw
    chunk = 2000
    assert per_w % chunk == 0 and chunk % 8 == 0
    nch = per_w // chunk
    mesh = plsc.VectorSubcoreMesh(core_axis_name="c", subcore_axis_name="s")

    @functools.partial(
        pl.kernel,
        out_type=jax.ShapeDtypeStruct((total_idx, 16), F32),
        mesh=mesh,
        scratch_types=[
            pltpu.VMEM((chunk,), jnp.int32),
            pltpu.VMEM((chunk,), jnp.int32),
            pltpu.VMEM((chunk, 16), F32),
            pltpu.VMEM((chunk, 16), F32),
            pltpu.SemaphoreType.DMA,
            pltpu.SemaphoreType.DMA,
        ],
        compiler_params=pltpu.CompilerParams(use_tc_tiling_on_sc=False),
    )
    def sc_gather(p_hbm, idx_hbm, g_hbm, idx0, idx1, buf0, buf1, sem0, sem1):
        wid = lax.axis_index("s") * nc + lax.axis_index("c")
        base = wid * per_w
        idxs, bufs, sems = [idx0, idx1], [buf0, buf1], [sem0, sem1]
        copies = [None, None]
        pltpu.sync_copy(idx_hbm.at[pl.ds(base, chunk)], idx0)
        copies[0] = pltpu.async_copy(p_hbm.at[idx0], buf0, sem0)
        for c in range(nch):
            cur = c & 1
            nxt = (c + 1) & 1
            if c + 1 < nch:
                pltpu.sync_copy(
                    idx_hbm.at[pl.ds(base + (c + 1) * chunk, chunk)], idxs[nxt]
                )
                copies[nxt] = pltpu.async_copy(
                    p_hbm.at[idxs[nxt]], bufs[nxt], sems[nxt]
                )
            copies[cur].wait()
            pltpu.sync_copy(bufs[cur], g_hbm.at[pl.ds(base + c * chunk, chunk)])

    return sc_gather


def _k1_body(atom_ref, ws_ref, wn_ref, bb_ref, s_ref, p_ref):
    a = atom_ref[...]
    s_ref[...] = (
        jnp.dot(a, ws_ref[...], preferred_element_type=F32) + bb_ref[...]
    )
    p_ref[...] = jnp.dot(a, wn_ref[...], preferred_element_type=F32)


def _k2_body(nb, g_ref, bond_ref, s_ref, w128_ref, e_ref, et_ref,
             bf_ref, pooled_ref, stats_ref):
    b = jnp.dot(bond_ref[...], w128_ref[...], preferred_element_type=F32)
    s128 = jnp.dot(s_ref[...], e_ref[...], preferred_element_type=F32)
    sfull = jnp.broadcast_to(s128[:, None, :], (nb, 4, 128)).reshape(nb * 4, 128)
    bf = jnp.tanh(g_ref[...] + b + sfull)
    bf_ref[...] = bf
    sum4 = jnp.sum(bf.reshape(nb, 4, 128), axis=1)
    pooled_ref[...] = jnp.dot(
        sum4, et_ref[...], preferred_element_type=F32
    ) * (1.0 / 32.0)
    part = jnp.concatenate(
        [jnp.sum(bf, axis=0, keepdims=True),
         jnp.sum(bf * bf, axis=0, keepdims=True)], axis=0)

    @pl.when(pl.program_id(0) == 0)
    def _():
        stats_ref[...] = part

    @pl.when(pl.program_id(0) != 0)
    def _():
        stats_ref[...] += part


def _k3_body(bf_ref, pooled_ref, atom_ref, wap_ref, waa_ref, ba_ref,
             aff16_ref, aff128_ref, bfout_ref, af_ref, stats_ref):
    a1_16 = aff16_ref[0:1, :]
    c1_16 = aff16_ref[1:2, :]
    a1_128 = aff128_ref[0:1, :]
    c1_128 = aff128_ref[1:2, :]
    bfout_ref[...] = bf_ref[...] * a1_128 + c1_128
    pooled_n = pooled_ref[...] * a1_16 + c1_16
    h = (
        jnp.dot(pooled_n, wap_ref[...], preferred_element_type=F32)
        + jnp.dot(atom_ref[...], waa_ref[...], preferred_element_type=F32)
        + ba_ref[...]
    )
    af = jnp.tanh(h)
    af_ref[...] = af
    part = jnp.concatenate(
        [jnp.sum(af, axis=0, keepdims=True),
         jnp.sum(af * af, axis=0, keepdims=True)], axis=0)

    @pl.when(pl.program_id(0) == 0)
    def _():
        stats_ref[...] = part

    @pl.when(pl.program_id(0) != 0)
    def _():
        stats_ref[...] += part


def _k4_body(af_ref, aff_ref, out_ref):
    out_ref[...] = af_ref[...] * aff_ref[0:1, :] + aff_ref[1:2, :]


def _affine_from_stats(stats16, cnt, gamma, beta):
    mu = stats16[0] / cnt
    var = stats16[1] / cnt - mu * mu
    a = gamma * lax.rsqrt(var + EPS)
    c = beta - mu * a
    return a, c


def kernel(nbr_idx, atom_fea, bond_fea, Wb, bb, Wa, ba, g1, be1, g2, be2):
    n, nnn = nbr_idx.shape
    a_fea = atom_fea.shape[1]
    b_fea = bond_fea.shape[2]
    assert (n, nnn, a_fea, b_fea) == (10000, 32, 128, 16)
    tot = n * nnn
    rows_p = tot * b_fea // 128
    nb = 1000
    grid = n // nb
    rbp = nb * nnn * b_fea // 128

    idx = nbr_idx.reshape(-1).astype(jnp.int32)
    Wb_self, Wb_nbr, Wb_bond = Wb[:a_fea], Wb[a_fea:2 * a_fea], Wb[2 * a_fea:]
    W128 = jnp.kron(jnp.eye(8, dtype=F32), Wb_bond)
    E = (jnp.arange(128)[None, :] % 16 == jnp.arange(16)[:, None]).astype(F32)
    bond_p = bond_fea.reshape(rows_p, 128)

    s_tab, p_tab = pl.pallas_call(
        _k1_body,
        grid=(grid,),
        in_specs=[
            pl.BlockSpec((nb, a_fea), lambda i: (i, 0)),
            pl.BlockSpec((a_fea, b_fea), lambda i: (0, 0)),
            pl.BlockSpec((a_fea, b_fea), lambda i: (0, 0)),
            pl.BlockSpec((1, b_fea), lambda i: (0, 0)),
        ],
        out_specs=[
            pl.BlockSpec((nb, b_fea), lambda i: (i, 0)),
            pl.BlockSpec((nb, b_fea), lambda i: (i, 0)),
        ],
        out_shape=[
            jax.ShapeDtypeStruct((n, b_fea), F32),
            jax.ShapeDtypeStruct((n, b_fea), F32),
        ],
    )(atom_fea, Wb_self, Wb_nbr, bb.reshape(1, b_fea))

    g_rows = _make_sc_gather(tot)(p_tab, idx)
    g_p = g_rows.reshape(rows_p, 128)

    bf_raw_p, pooled, stats1 = pl.pallas_call(
        functools.partial(_k2_body, nb),
        grid=(grid,),
        in_specs=[
            pl.BlockSpec((rbp, 128), lambda i: (i, 0)),
            pl.BlockSpec((rbp, 128), lambda i: (i, 0)),
            pl.BlockSpec((nb, b_fea), lambda i: (i, 0)),
            pl.BlockSpec((128, 128), lambda i: (0, 0)),
            pl.BlockSpec((b_fea, 128), lambda i: (0, 0)),
            pl.BlockSpec((128, b_fea), lambda i: (0, 0)),
        ],
        out_specs=[
            pl.BlockSpec((rbp, 128), lambda i: (i, 0)),
            pl.BlockSpec((nb, b_fea), lambda i: (i, 0)),
            pl.BlockSpec((2, 128), lambda i: (0, 0)),
        ],
        out_shape=[
            jax.ShapeDtypeStruct((rows_p, 128), F32),
            jax.ShapeDtypeStruct((n, b_fea), F32),
            jax.ShapeDtypeStruct((2, 128), F32),
        ],
    )(g_p, bond_p, s_tab, W128, E, E.T)

    st16 = stats1.reshape(2, 8, b_fea).sum(axis=1)
    a1, c1 = _affine_from_stats(st16, float(tot), g1, be1)
    aff16 = jnp.stack([a1, c1])
    aff128 = jnp.stack([jnp.tile(a1, 8), jnp.tile(c1, 8)])

    bf_out_p, af_raw, stats2 = pl.pallas_call(
        _k3_body,
        grid=(grid,),
        in_specs=[
            pl.BlockSpec((rbp, 128), lambda i: (i, 0)),
            pl.BlockSpec((nb, b_fea), lambda i: (i, 0)),
            pl.BlockSpec((nb, a_fea), lambda i: (i, 0)),
            pl.BlockSpec((b_fea, a_fea), lambda i: (0, 0)),
            pl.BlockSpec((a_fea, a_fea), lambda i: (0, 0)),
            pl.BlockSpec((1, a_fea), lambda i: (0, 0)),
            pl.BlockSpec((2, b_fea), lambda i: (0, 0)),
            pl.BlockSpec((2, 128), lambda i: (0, 0)),
        ],
        out_specs=[
            pl.BlockSpec((rbp, 128), lambda i: (i, 0)),
            pl.BlockSpec((nb, a_fea), lambda i: (i, 0)),
            pl.BlockSpec((2, a_fea), lambda i: (0, 0)),
        ],
        out_shape=[
            jax.ShapeDtypeStruct((rows_p, 128), F32),
            jax.ShapeDtypeStruct((n, a_fea), F32),
            jax.ShapeDtypeStruct((2, a_fea), F32),
        ],
    )(bf_raw_p, pooled, atom_fea, Wa[:b_fea], Wa[b_fea:],
      ba.reshape(1, a_fea), aff16, aff128)

    a2, c2 = _affine_from_stats(stats2, float(n), g2, be2)

    af_out = pl.pallas_call(
        _k4_body,
        grid=(grid,),
        in_specs=[
            pl.BlockSpec((nb, a_fea), lambda i: (i, 0)),
            pl.BlockSpec((2, a_fea), lambda i: (0, 0)),
        ],
        out_specs=pl.BlockSpec((nb, a_fea), lambda i: (i, 0)),
        out_shape=jax.ShapeDtypeStruct((n, a_fea), F32),
    )(af_raw, jnp.stack([a2, c2]))

    return af_out, bf_out_p.reshape(n, nnn, b_fea)


# speedup vs baseline: 12.2070x; 3.8841x over previous
import functools

import jax
import jax.numpy as jnp
from jax import lax
from jax.experimental import pallas as pl
from jax.experimental.pallas import tpu as pltpu
from jax.experimental.pallas import tpu_sc as plsc

EPS = 1e-5
F32 = jnp.float32


def _make_sc_gather(total_idx, n_rows):
    info = plsc.get_sparse_core_info()
    nc, ns = info.num_cores, info.num_subcores
    nw = nc * ns
    assert total_idx % nw == 0
    per_w = total_idx // nw
    chunk = 2000
    assert per_w % chunk == 0 and chunk % 8 == 0
    nch = per_w // chunk
    mesh = plsc.VectorSubcoreMesh(core_axis_name="c", subcore_axis_name="s")

    @functools.partial(
        pl.kernel,
        out_type=jax.ShapeDtypeStruct((total_idx, 16), F32),
        mesh=mesh,
        scratch_types=[
            pltpu.VMEM_SHARED((n_rows, 16), F32),
            pltpu.VMEM((chunk,), jnp.int32),
            pltpu.VMEM((chunk,), jnp.int32),
            pltpu.VMEM((chunk, 16), F32),
            pltpu.VMEM((chunk, 16), F32),
            pltpu.SemaphoreType.DMA,
            pltpu.SemaphoreType.DMA,
        ],
        compiler_params=pltpu.CompilerParams(use_tc_tiling_on_sc=False),
    )
    def sc_gather(p_hbm, idx_hbm, g_hbm, p_sh, idx0, idx1, buf0, buf1,
                  sem0, sem1):
        @pl.when(lax.axis_index("s") == 0)
        def _():
            pltpu.sync_copy(p_hbm, p_sh)

        plsc.subcore_barrier()
        wid = lax.axis_index("s") * nc + lax.axis_index("c")
        base = wid * per_w
        idxs, bufs, sems = [idx0, idx1], [buf0, buf1], [sem0, sem1]
        copies = [None, None]
        pltpu.sync_copy(idx_hbm.at[pl.ds(base, chunk)], idx0)
        copies[0] = pltpu.async_copy(p_sh.at[idx0], buf0, sem0)
        for c in range(nch):
            cur = c & 1
            nxt = (c + 1) & 1
            if c + 1 < nch:
                pltpu.sync_copy(
                    idx_hbm.at[pl.ds(base + (c + 1) * chunk, chunk)], idxs[nxt]
                )
                copies[nxt] = pltpu.async_copy(
                    p_sh.at[idxs[nxt]], bufs[nxt], sems[nxt]
                )
            copies[cur].wait()
            pltpu.sync_copy(bufs[cur], g_hbm.at[pl.ds(base + c * chunk, chunk)])

    return sc_gather


def _k1_body(atom_ref, ws_ref, wn_ref, bb_ref, s_ref, p_ref):
    a = atom_ref[...]
    s_ref[...] = (
        jnp.dot(a, ws_ref[...], preferred_element_type=F32) + bb_ref[...]
    )
    p_ref[...] = jnp.dot(a, wn_ref[...], preferred_element_type=F32)


def _mega_body(nb, n, cnt1,
               g_ref, bond_ref, s_ref, atom_ref, w512_ref, e512_ref,
               epool_ref, wap_ref, waa_ref, ba_ref, g1be1_ref, g2be2_ref,
               bfout_ref, afout_ref,
               bf_s, pooled_s, af_s, st1_s, st2_s):
    p = pl.program_id(0)
    i = pl.program_id(1)
    row = lax.broadcasted_iota(jnp.int32, (nb, 1), 0) + i * nb
    sl = pl.ds(i * nb, nb)

    @pl.when(p == 0)
    def _():
        b512 = lax.dot_general(
            bond_ref[...], w512_ref[...], (((0,), (0,)), ((), ())),
            preferred_element_type=F32)
        s512 = jnp.dot(s_ref[...], e512_ref[...], preferred_element_type=F32)
        g512 = g_ref[...].reshape(nb, 512)
        bf = jnp.tanh(g512 + b512 + s512)
        bf_s[sl, :] = bf.astype(jnp.bfloat16)
        bfm = jnp.where(row < n, bf, 0.0)
        pooled_s[sl, :] = jnp.dot(bfm, epool_ref[...],
                                  preferred_element_type=F32)
        part = jnp.concatenate(
            [jnp.sum(bfm, axis=0, keepdims=True),
             jnp.sum(bfm * bfm, axis=0, keepdims=True)], axis=0)

        @pl.when(i == 0)
        def _():
            st1_s[...] = part

        @pl.when(i != 0)
        def _():
            st1_s[...] += part

    @pl.when(p == 1)
    def _():
        st16 = lax.dot_general(
            st1_s[...], e512_ref[...], (((1,), (1,)), ((), ())),
            preferred_element_type=F32)
        mu = st16[0:1, :] * (1.0 / cnt1)
        var = st16[1:2, :] * (1.0 / cnt1) - mu * mu
        a1 = g1be1_ref[0:1, :] * lax.rsqrt(var + EPS)
        c1 = g1be1_ref[1:2, :] - mu * a1
        a1_512 = jnp.dot(a1, e512_ref[...], preferred_element_type=F32)
        c1_512 = jnp.dot(c1, e512_ref[...], preferred_element_type=F32)
        bf = bf_s[sl, :].astype(F32)
        bfout_ref[...] = jnp.transpose(bf * a1_512 + c1_512, (1, 0))
        pooled_n = pooled_s[sl, :] * a1 + c1
        h = (
            jnp.dot(pooled_n, wap_ref[...], preferred_element_type=F32)
            + jnp.dot(atom_ref[...], waa_ref[...], preferred_element_type=F32)
            + ba_ref[...]
        )
        af = jnp.tanh(h)
        af_s[sl, :] = af
        afm = jnp.where(row < n, af, 0.0)
        part = jnp.concatenate(
            [jnp.sum(afm, axis=0, keepdims=True),
             jnp.sum(afm * afm, axis=0, keepdims=True)], axis=0)

        @pl.when(i == 0)
        def _():
            st2_s[...] = part

        @pl.when(i != 0)
        def _():
            st2_s[...] += part

    @pl.when(p == 2)
    def _():
        mu = st2_s[0:1, :] * (1.0 / n)
        var = st2_s[1:2, :] * (1.0 / n) - mu * mu
        a2 = g2be2_ref[0:1, :] * lax.rsqrt(var + EPS)
        c2 = g2be2_ref[1:2, :] - mu * a2
        afout_ref[...] = af_s[sl, :] * a2 + c2


def kernel(nbr_idx, atom_fea, bond_fea, Wb, bb, Wa, ba, g1, be1, g2, be2):
    n, nnn = nbr_idx.shape
    a_fea = atom_fea.shape[1]
    b_fea = bond_fea.shape[2]
    assert (n, nnn, a_fea, b_fea) == (10000, 32, 128, 16)
    tot = n * nnn
    rows_p = tot * b_fea // 128
    nb = 1000
    grid = n // nb
    nbk = 2048
    gridk = -(-n // nbk)
    rbk = nbk * nnn * b_fea // 128

    idx = nbr_idx.reshape(-1).astype(jnp.int32)
    Wb_self, Wb_nbr, Wb_bond = Wb[:a_fea], Wb[a_fea:2 * a_fea], Wb[2 * a_fea:]
    jc = nnn * b_fea
    W512 = jnp.kron(jnp.eye(nnn, dtype=F32), Wb_bond)
    E512 = (jnp.arange(jc)[None, :] % b_fea
            == jnp.arange(b_fea)[:, None]).astype(F32)
    Epool = E512.T * (1.0 / nnn)
    bond_t = jnp.transpose(bond_fea, (1, 2, 0)).reshape(jc, n)

    s_tab, p_tab = pl.pallas_call(
        _k1_body,
        grid=(grid,),
        in_specs=[
            pl.BlockSpec((nb, a_fea), lambda i: (i, 0)),
            pl.BlockSpec((a_fea, b_fea), lambda i: (0, 0)),
            pl.BlockSpec((a_fea, b_fea), lambda i: (0, 0)),
            pl.BlockSpec((1, b_fea), lambda i: (0, 0)),
        ],
        out_specs=[
            pl.BlockSpec((nb, b_fea), lambda i: (i, 0)),
            pl.BlockSpec((nb, b_fea), lambda i: (i, 0)),
        ],
        out_shape=[
            jax.ShapeDtypeStruct((n, b_fea), F32),
            jax.ShapeDtypeStruct((n, b_fea), F32),
        ],
    )(atom_fea, Wb_self, Wb_nbr, bb.reshape(1, b_fea))

    g_rows = _make_sc_gather(tot, n)(p_tab, idx)
    g_p = g_rows.reshape(rows_p, 128)

    last = gridk - 1
    g_map = lambda p, i: (jnp.where(p == 0, i, last), 0)
    bond_map = lambda p, i: (0, jnp.where(p == 0, i, last))
    s_map = lambda p, i: (jnp.where(p == 0, i, last), 0)
    atom_map = lambda p, i: (jnp.where(p == 0, 0, jnp.where(p == 1, i, last)), 0)
    bfout_map = lambda p, i: (0, jnp.where(p == 0, 0, jnp.where(p == 1, i, last)))
    afout_map = lambda p, i: (jnp.where(p == 2, i, 0), 0)
    const2 = lambda p, i: (0, 0)
    npad = gridk * nbk

    bf_out_t, af_out = pl.pallas_call(
        functools.partial(_mega_body, nbk, n, float(tot)),
        grid=(3, gridk),
        in_specs=[
            pl.BlockSpec((rbk, 128), g_map),
            pl.BlockSpec((jc, nbk), bond_map),
            pl.BlockSpec((nbk, b_fea), s_map),
            pl.BlockSpec((nbk, a_fea), atom_map),
            pl.BlockSpec((jc, jc), const2),
            pl.BlockSpec((b_fea, jc), const2),
            pl.BlockSpec((jc, b_fea), const2),
            pl.BlockSpec((b_fea, a_fea), const2),
            pl.BlockSpec((a_fea, a_fea), const2),
            pl.BlockSpec((1, a_fea), const2),
            pl.BlockSpec((2, b_fea), const2),
            pl.BlockSpec((2, a_fea), const2),
        ],
        out_specs=[
            pl.BlockSpec((jc, nbk), bfout_map),
            pl.BlockSpec((nbk, a_fea), afout_map),
        ],
        out_shape=[
            jax.ShapeDtypeStruct((jc, n), F32),
            jax.ShapeDtypeStruct((n, a_fea), F32),
        ],
        scratch_shapes=[
            pltpu.VMEM((npad, jc), jnp.bfloat16),
            pltpu.VMEM((npad, b_fea), F32),
            pltpu.VMEM((npad, a_fea), F32),
            pltpu.VMEM((2, jc), F32),
            pltpu.VMEM((2, a_fea), F32),
        ],
        compiler_params=pltpu.CompilerParams(
            vmem_limit_bytes=100 * 1024 * 1024),
    )(g_p, bond_t, s_tab, atom_fea, W512, E512, Epool,
      Wa[:b_fea], Wa[b_fea:], ba.reshape(1, a_fea),
      jnp.stack([g1, be1]), jnp.stack([g2, be2]))

    bf_out = bf_out_t.reshape(nnn, b_fea, n).transpose(2, 0, 1)
    return af_out, bf_out
